# Initial kernel scaffold; baseline (speedup 1.0000x reference)
import jax
import jax.numpy as jnp
from jax.experimental import pallas as pl


def kernel(descriptors, scores, scores_dense, imgs):
    raise NotImplementedError("write your pallas kernel here")


# speedup vs baseline: 49.8805x; 49.8805x over previous
import math

import jax
import jax.numpy as jnp
import numpy as np
from jax.experimental import pallas as pl

_B, _H, _W = 4, 224, 224
_R = 8
_NUM = 200
_HB, _WB = _H // _R, _W // _R
_N_DESC, _D_DESC = 256, 128


def _gauss_taps(ksize=7, sigma=1.0):
    x = np.arange(ksize, dtype=np.float64) - (ksize - 1) / 2.0
    g = np.exp(-(x ** 2) / (2.0 * sigma ** 2))
    g = g / g.sum()
    return [float(v) for v in g]


def _rpad(x, axis, p):
    n = x.shape[axis]
    parts = [jax.lax.slice_in_dim(x, p - k, p - k + 1, axis=axis)
             for k in range(p)]
    parts.append(x)
    parts += [jax.lax.slice_in_dim(x, n - 2 - k, n - 1 - k, axis=axis)
              for k in range(p)]
    return jnp.concatenate(parts, axis=axis)


def _conv1(x, taps, axis):
    p = len(taps) // 2
    n = x.shape[axis]
    xp = _rpad(x, axis, p)
    acc = None
    for k, w in enumerate(taps):
        if w == 0.0:
            continue
        s = jax.lax.slice_in_dim(xp, k, k + n, axis=axis)
        term = s if w == 1.0 else s * w
        acc = term if acc is None else acc + term
    return acc


def _maxpool1(x, axis, ks=5):
    p = ks // 2
    n = x.shape[axis]
    shp = list(x.shape)
    shp[axis] = p
    pad = jnp.full(shp, -jnp.inf, x.dtype)
    xp = jnp.concatenate([pad, x, pad], axis=axis)
    acc = None
    for k in range(ks):
        s = jax.lax.slice_in_dim(xp, k, k + n, axis=axis)
        acc = s if acc is None else jnp.maximum(acc, s)
    return acc


def _loss_kernel(imgs_ref, sd_ref, desc_ref, out_ref):
    imgs = imgs_ref[...]
    gray = (0.299 * imgs[:, 0] + 0.587 * imgs[:, 1]
            + 0.114 * imgs[:, 2])

    dx = _conv1(_conv1(gray, [-1.0, 0.0, 1.0], axis=2),
                [1.0, 2.0, 1.0], axis=1) * 0.125
    dy = _conv1(_conv1(gray, [-1.0, 0.0, 1.0], axis=1),
                [1.0, 2.0, 1.0], axis=2) * 0.125

    g7 = _gauss_taps()

    def blur(z):
        return _conv1(_conv1(z, g7, axis=2), g7, axis=1)

    dx2 = blur(dx * dx)
    dy2 = blur(dy * dy)
    dxy = blur(dx * dy)
    det = dx2 * dy2 - dxy * dxy
    trace = dx2 + dy2
    e = 0.5 * (trace - jnp.sqrt(jnp.maximum(trace * trace - 4.0 * det, 0.0)
                                + 1e-12))

    mp = _maxpool1(_maxpool1(e, axis=1), axis=2)
    nms = e * (e == mp).astype(e.dtype)

    xh = jnp.max(nms.reshape(_B, _HB, _R, _W), axis=2)
    xt = jnp.swapaxes(xh, 1, 2)
    c_t = jnp.max(xt.reshape(_B, _WB, _R, _HB), axis=2)
    bm_t = jnp.broadcast_to(c_t[:, :, None, :],
                            (_B, _WB, _R, _HB)).reshape(_B, _W, _HB)
    bm_h = jnp.swapaxes(bm_t, 1, 2)
    bmax = jnp.broadcast_to(bm_h[:, :, None, :],
                            (_B, _HB, _R, _W)).reshape(_B, _H, _W)

    cand = jnp.maximum(c_t, 0.0).reshape(_B, _WB * _HB)
    cbits = jax.lax.bitcast_convert_type(cand, jnp.int32)

    lo = jnp.ones((_B, 1), jnp.int32)
    hi = jnp.full((_B, 1), jnp.int32(0x7F7FFFFF))

    def bs_body(_, carry):
        lo, hi = carry
        mid = lo + (hi - lo + 1) // 2
        cnt = jnp.sum((cbits >= mid).astype(jnp.int32), axis=1,
                      keepdims=True)
        ok = cnt >= _NUM
        return jnp.where(ok, mid, lo), jnp.where(ok, hi, mid - 1)

    lo, _ = jax.lax.fori_loop(0, 31, bs_body, (lo, hi))
    thr = jax.lax.bitcast_convert_type(lo, jnp.float32)
    thr3 = thr.reshape(_B, 1, 1)

    s = sd_ref[...].reshape(_B, _H, _W)
    a_sum = jnp.sum(jnp.maximum(s, 0.0)
                    + jnp.log(1.0 + jnp.exp(-jnp.abs(s))))
    surv = (nms > 0.0) & (nms == bmax)
    sel_sum = jnp.sum(jnp.where(surv & (nms >= thr3), s, 0.0))

    d = desc_ref[...]
    cos_sum = jnp.float32(0.0)
    for b in range(_B):
        db = d[b]
        sq = jnp.sum(db * db, axis=1, keepdims=True)
        nr = jnp.sqrt(sq)
        denom = jnp.maximum(nr * jnp.transpose(nr), 1e-8)
        dots = jax.lax.dot_general(db, db, (((1,), (1,)), ((), ())),
                                   preferred_element_type=jnp.float32)
        cos_sum = cos_sum + jnp.sum(jnp.maximum(dots, 0.0) / denom)

    npix = float(_B * _H * _W)
    ncos = float(_B * _N_DESC * _N_DESC)
    loss = (a_sum - sel_sum) / npix + cos_sum / ncos
    out_ref[...] = loss.reshape(1, 1)


def kernel(descriptors, scores, scores_dense, imgs):
    del scores
    out = pl.pallas_call(
        _loss_kernel,
        out_shape=jax.ShapeDtypeStruct((1, 1), jnp.float32),
    )(imgs, scores_dense, descriptors)
    return out[0, 0]
